# baseline (speedup 1.0000x reference)
import functools

import jax
import jax.numpy as jnp
import numpy as np
from jax import lax
from jax.experimental import pallas as pl
from jax.experimental.pallas import tpu as pltpu
from jax.experimental.pallas import tpu_sc as plsc

_T = 8192
_E = 64
_D = 2048
_EPS = 1e-20
_BT = 512

_NC = 2
_NS = 16
_NW = _NC * _NS
_BPW = _T // _NW
_SUB = 16
_NSUB = _BPW // _SUB
_NBUF = 3


@functools.lru_cache(maxsize=1)
def _gumbel_const() -> np.ndarray:
    with jax.ensure_compile_time_eval():
        uni = jax.random.uniform(jax.random.key(42), (_T, _E), dtype=jnp.float32)
        g = -jnp.log(-jnp.log(uni + _EPS) + _EPS)
        return np.asarray(g)


def _tc_body(u_ref, i_ref, g_ref, pw_ref, rw_ref, pre_ref, idx_ref):
    x = u_ref[...] + i_ref[...]
    w = pw_ref[...] + rw_ref[...]
    logits = jax.lax.dot_general(
        x, w, (((1,), (1,)), ((), ())),
        preferred_element_type=jnp.float32) * 0.5
    y = logits + g_ref[...]
    mx = jnp.max(y, axis=1, keepdims=True)
    iota = jax.lax.broadcasted_iota(jnp.int32, (_BT, _E), 1)
    idx = jnp.min(jnp.where(y == mx, iota, _E), axis=1, keepdims=True)
    pre_ref[...] = (iota == idx).astype(jnp.float32)
    idx_ref[...] = idx[:, 0]


def _sc_body(tblr_hbm, tbln_hbm, idx_hbm, re_hbm, nm_hbm,
             idx_v, b0, b1, b2, gs0, gs1, gs2, ss0, ss1, ss2):
    c = lax.axis_index("c")
    s = lax.axis_index("s")
    wid = s * _NC + c
    base = wid * _BPW
    pltpu.sync_copy(idx_hbm.at[pl.ds(base, _BPW)], idx_v)

    bufs = (b0, b1, b2)
    gsem = (gs0, gs1, gs2)
    ssem = (ss0, ss1, ss2)
    nstep = 2 * _NSUB

    def start_gather(k):
        i = k % _NBUF
        tbl = tblr_hbm if k < _NSUB else tbln_hbm
        off = (k % _NSUB) * _SUB
        return pltpu.async_copy(
            tbl.at[idx_v.at[pl.ds(off, _SUB)]], bufs[i], gsem[i])

    gd = {}
    sd = {}
    for k in range(_NBUF):
        gd[k] = start_gather(k)
    for k in range(nstep):
        i = k % _NBUF
        out = re_hbm if k < _NSUB else nm_hbm
        row = base + (k % _NSUB) * _SUB
        gd[k].wait()
        sd[k] = pltpu.async_copy(bufs[i], out.at[pl.ds(row, _SUB)], ssem[i])
        nk = k + _NBUF
        if nk < nstep:
            sd[k].wait()
            gd[nk] = start_gather(nk)
    for k in range(nstep - _NBUF, nstep):
        sd[k].wait()


def kernel(u_e, i_e, pref_w, rel_w, pref_norm_w, norm_w):
    g = jnp.asarray(_gumbel_const())

    grid = (_T // _BT,)
    tok = lambda t: (t, 0)
    tok1 = lambda t: (t,)
    fixed = lambda t: (0, 0)
    pre, idx = pl.pallas_call(
        _tc_body,
        grid=grid,
        in_specs=[
            pl.BlockSpec((_BT, _D), tok),
            pl.BlockSpec((_BT, _D), tok),
            pl.BlockSpec((_BT, _E), tok),
            pl.BlockSpec((_E, _D), fixed),
            pl.BlockSpec((_E, _D), fixed),
        ],
        out_specs=[
            pl.BlockSpec((_BT, _E), tok),
            pl.BlockSpec((_BT,), tok1),
        ],
        out_shape=[
            jax.ShapeDtypeStruct((_T, _E), jnp.float32),
            jax.ShapeDtypeStruct((_T,), jnp.int32),
        ],
        compiler_params=pltpu.CompilerParams(
            dimension_semantics=("arbitrary",),
        ),
    )(u_e, i_e, g, pref_w, rel_w)

    tbl_r = (pref_w + rel_w) * 0.5
    tbl_n = (pref_norm_w + norm_w) * 0.5

    mesh = plsc.VectorSubcoreMesh(core_axis_name="c", subcore_axis_name="s")
    sc_gather = pl.kernel(
        _sc_body,
        out_type=[
            jax.ShapeDtypeStruct((_T, _D), jnp.float32),
            jax.ShapeDtypeStruct((_T, _D), jnp.float32),
        ],
        mesh=mesh,
        scratch_types=(
            [pltpu.VMEM((_BPW,), jnp.int32)]
            + [pltpu.VMEM((_SUB, _D), jnp.float32)] * _NBUF
            + [pltpu.SemaphoreType.DMA] * (2 * _NBUF)
        ),
    )
    re, nm = sc_gather(tbl_r, tbl_n, idx)
    return (pre, re, nm)


# speedup vs baseline: 1.6697x; 1.6697x over previous
import functools

import jax
import jax.numpy as jnp
import numpy as np
from jax import lax
from jax.experimental import pallas as pl
from jax.experimental.pallas import tpu as pltpu
from jax.experimental.pallas import tpu_sc as plsc

_T = 8192
_E = 64
_D = 2048
_EPS = 1e-20
_BT = 512
_NCHUNK = 4
_CT = _T // _NCHUNK
_NBLK = _CT // _BT

_NC = 2
_NS = 16
_NW = _NC * _NS
_BPW = _CT // _NW
_SUB = 16


@functools.lru_cache(maxsize=1)
def _gumbel_const() -> np.ndarray:
    with jax.ensure_compile_time_eval():
        uni = jax.random.uniform(jax.random.key(42), (_T, _E), dtype=jnp.float32)
        g = -jnp.log(-jnp.log(uni + _EPS) + _EPS)
        return np.asarray(g)


def _tc_body_first(u_ref, i_ref, g_ref, pw_ref, rw_ref, pnw_ref, nw_ref,
                   pre_ref, idx_ref, nm_ref):
    _tc_compute(u_ref, i_ref, g_ref, pw_ref, rw_ref, pnw_ref, nw_ref,
                pre_ref, idx_ref, nm_ref)


def _tc_body_chain(u_ref, i_ref, g_ref, pw_ref, rw_ref, pnw_ref, nw_ref,
                   nm_prev_ref, pre_ref, idx_ref, nm_ref):
    del nm_prev_ref
    _tc_compute(u_ref, i_ref, g_ref, pw_ref, rw_ref, pnw_ref, nw_ref,
                pre_ref, idx_ref, nm_ref)


def _tc_compute(u_ref, i_ref, g_ref, pw_ref, rw_ref, pnw_ref, nw_ref,
                pre_ref, idx_ref, nm_ref):
    x = u_ref[...] + i_ref[...]
    w = pw_ref[...] + rw_ref[...]
    logits = jax.lax.dot_general(
        x, w, (((1,), (1,)), ((), ())),
        preferred_element_type=jnp.float32) * 0.5
    y = logits + g_ref[...]
    mx = jnp.max(y, axis=1, keepdims=True)
    iota = jax.lax.broadcasted_iota(jnp.int32, (_BT, _E), 1)
    idx = jnp.min(jnp.where(y == mx, iota, _E), axis=1, keepdims=True)
    onehot = (iota == idx).astype(jnp.float32)
    pre_ref[...] = onehot
    idx_ref[...] = idx[:, 0]
    wn = pnw_ref[...] + nw_ref[...]
    nm_ref[...] = jax.lax.dot_general(
        onehot, wn, (((1,), (0,)), ((), ())),
        preferred_element_type=jnp.float32) * 0.5


def _sc_body(chunk, tblr_hbm, idx_hbm, re_hbm, ts_r, idx_v, sem_r):
    c = lax.axis_index("c")
    s = lax.axis_index("s")
    wid = s * _NC + c
    base = chunk * _CT + wid * _BPW

    @pl.when(s == 0)
    def _stage():
        pltpu.sync_copy(tblr_hbm, ts_r)

    pltpu.sync_copy(idx_hbm.at[pl.ds(wid * _BPW, _BPW)], idx_v)
    plsc.subcore_barrier()

    prev = None
    for g in range(_BPW // _SUB):
        iv = idx_v[pl.ds(g * _SUB, _SUB)]
        row0 = base + g * _SUB
        cur = []
        for j in range(_SUB):
            i = iv[j]
            cur.append(pltpu.async_copy(
                ts_r.at[pl.ds(i, 1)], re_hbm.at[pl.ds(row0 + j, 1)], sem_r))
        if prev is not None:
            for d in prev:
                d.wait()
        prev = cur
    for d in prev:
        d.wait()


def _make_tc_call(k):
    tok = lambda t: (t, 0)
    tok1 = lambda t: (t,)
    nm_map = lambda t: (t + k * _NBLK, 0)
    fixed = lambda t: (0, 0)
    in_specs = [
        pl.BlockSpec((_BT, _D), lambda t, _k=k: (t + _k * _NBLK, 0)),
        pl.BlockSpec((_BT, _D), lambda t, _k=k: (t + _k * _NBLK, 0)),
        pl.BlockSpec((_BT, _E), lambda t, _k=k: (t + _k * _NBLK, 0)),
        pl.BlockSpec((_E, _D), fixed),
        pl.BlockSpec((_E, _D), fixed),
        pl.BlockSpec((_E, _D), fixed),
        pl.BlockSpec((_E, _D), fixed),
    ]
    out_specs = [
        pl.BlockSpec((_BT, _E), tok),
        pl.BlockSpec((_BT,), tok1),
        pl.BlockSpec((_BT, _D), nm_map),
    ]
    out_shape = [
        jax.ShapeDtypeStruct((_CT, _E), jnp.float32),
        jax.ShapeDtypeStruct((_CT,), jnp.int32),
        jax.ShapeDtypeStruct((_T, _D), jnp.float32),
    ]
    params = pltpu.CompilerParams(dimension_semantics=("arbitrary",))
    if k == 0:
        return pl.pallas_call(
            _tc_body_first, grid=(_NBLK,), in_specs=in_specs,
            out_specs=out_specs, out_shape=out_shape, compiler_params=params)
    return pl.pallas_call(
        _tc_body_chain, grid=(_NBLK,),
        in_specs=in_specs + [pl.BlockSpec(memory_space=pl.ANY)],
        out_specs=out_specs, out_shape=out_shape,
        input_output_aliases={7: 2},
        compiler_params=params)


def _make_sc_call(k, with_out):
    mesh = plsc.VectorSubcoreMesh(core_axis_name="c", subcore_axis_name="s")
    body = functools.partial(_sc_body, k)
    out_type = jax.ShapeDtypeStruct((_T, _D), jnp.float32) if with_out else []
    return pl.kernel(
        body,
        out_type=out_type,
        mesh=mesh,
        scratch_types=[
            pltpu.VMEM_SHARED((_E, _D), jnp.float32),
            pltpu.VMEM((_BPW,), jnp.int32),
            pltpu.SemaphoreType.DMA,
        ],
    )


def kernel(u_e, i_e, pref_w, rel_w, pref_norm_w, norm_w):
    g = jnp.asarray(_gumbel_const())
    tbl_r = (pref_w + rel_w) * 0.5

    pres = []
    idxs = []
    nm = None
    for k in range(_NCHUNK):
        call = _make_tc_call(k)
        if k == 0:
            pre_k, idx_k, nm = call(u_e, i_e, g, pref_w, rel_w,
                                    pref_norm_w, norm_w)
        else:
            pre_k, idx_k, nm = call(u_e, i_e, g, pref_w, rel_w,
                                    pref_norm_w, norm_w, nm)
        pres.append(pre_k)
        idxs.append(idx_k)

    re0 = _make_sc_call(0, True)(tbl_r, idxs[0])
    re_ref = jax.new_ref(re0)
    for k in range(1, _NCHUNK):
        _make_sc_call(k, False)(tbl_r, idxs[k], re_ref)
    re = re_ref[...]

    pre = jnp.concatenate(pres, axis=0)
    return (pre, re, nm)


# speedup vs baseline: 2.3158x; 1.3869x over previous
import functools

import jax
import jax.numpy as jnp
import numpy as np
from jax.experimental import pallas as pl
from jax.experimental.pallas import tpu as pltpu

_T = 8192
_E = 64
_D = 2048
_EPS = 1e-20
_BT = 512


@functools.lru_cache(maxsize=1)
def _gumbel_const() -> np.ndarray:
    with jax.ensure_compile_time_eval():
        uni = jax.random.uniform(jax.random.key(42), (_T, _E), dtype=jnp.float32)
        g = -jnp.log(-jnp.log(uni + _EPS) + _EPS)
        return np.asarray(g)


def _body(u_ref, i_ref, g_ref, pw_ref, rw_ref, pnw_ref, nw_ref,
          pre_ref, re_ref, nm_ref):
    x = u_ref[...] + i_ref[...]
    w = pw_ref[...] + rw_ref[...]
    logits = jax.lax.dot_general(
        x, w, (((1,), (1,)), ((), ())),
        preferred_element_type=jnp.float32) * 0.5
    y = logits + g_ref[...]
    mx = jnp.max(y, axis=1, keepdims=True)
    iota = jax.lax.broadcasted_iota(jnp.int32, (_BT, _E), 1)
    idx = jnp.min(jnp.where(y == mx, iota, _E), axis=1, keepdims=True)
    onehot = (iota == idx).astype(jnp.float32)
    pre_ref[...] = onehot
    re_ref[...] = jax.lax.dot_general(
        onehot, w, (((1,), (0,)), ((), ())),
        preferred_element_type=jnp.float32) * 0.5
    wn = pnw_ref[...] + nw_ref[...]
    nm_ref[...] = jax.lax.dot_general(
        onehot, wn, (((1,), (0,)), ((), ())),
        preferred_element_type=jnp.float32) * 0.5


def kernel(u_e, i_e, pref_w, rel_w, pref_norm_w, norm_w):
    g = jnp.asarray(_gumbel_const())

    grid = (_T // _BT,)
    tok = lambda t: (t, 0)
    fixed = lambda t: (0, 0)
    pre, re, nm = pl.pallas_call(
        _body,
        grid=grid,
        in_specs=[
            pl.BlockSpec((_BT, _D), tok),
            pl.BlockSpec((_BT, _D), tok),
            pl.BlockSpec((_BT, _E), tok),
            pl.BlockSpec((_E, _D), fixed),
            pl.BlockSpec((_E, _D), fixed),
            pl.BlockSpec((_E, _D), fixed),
            pl.BlockSpec((_E, _D), fixed),
        ],
        out_specs=[
            pl.BlockSpec((_BT, _E), tok),
            pl.BlockSpec((_BT, _D), tok),
            pl.BlockSpec((_BT, _D), tok),
        ],
        out_shape=[
            jax.ShapeDtypeStruct((_T, _E), jnp.float32),
            jax.ShapeDtypeStruct((_T, _D), jnp.float32),
            jax.ShapeDtypeStruct((_T, _D), jnp.float32),
        ],
        compiler_params=pltpu.CompilerParams(
            dimension_semantics=("arbitrary",),
        ),
    )(u_e, i_e, g, pref_w, rel_w, pref_norm_w, norm_w)
    return (pre, re, nm)


# speedup vs baseline: 2.3169x; 1.0005x over previous
import functools

import jax
import jax.numpy as jnp
import numpy as np
from jax.experimental import pallas as pl
from jax.experimental.pallas import tpu as pltpu

_T = 8192
_E = 64
_D = 2048
_EPS = 1e-20
_BT = 512


@functools.lru_cache(maxsize=1)
def _gumbel_const() -> np.ndarray:
    with jax.ensure_compile_time_eval():
        uni = jax.random.uniform(jax.random.key(42), (_T, _E), dtype=jnp.float32)
        g = -jnp.log(-jnp.log(uni + _EPS) + _EPS)
        return np.asarray(g)


def _body(u_ref, i_ref, g_ref, pw_ref, rw_ref, pnw_ref, nw_ref,
          pre_ref, re_ref, nm_ref):
    x = u_ref[...] + i_ref[...]
    w = pw_ref[...] + rw_ref[...]
    logits = jax.lax.dot_general(
        x, w, (((1,), (1,)), ((), ())),
        preferred_element_type=jnp.float32) * 0.5
    y = logits + g_ref[...]
    mx = jnp.max(y, axis=1, keepdims=True)
    iota = jax.lax.broadcasted_iota(jnp.int32, (_BT, _E), 1)
    idx = jnp.min(jnp.where(y == mx, iota, _E), axis=1, keepdims=True)
    onehot = (iota == idx).astype(jnp.float32)
    pre_ref[...] = onehot
    re_ref[...] = jax.lax.dot_general(
        onehot, w, (((1,), (0,)), ((), ())),
        preferred_element_type=jnp.float32) * 0.5
    wn = pnw_ref[...] + nw_ref[...]
    nm_ref[...] = jax.lax.dot_general(
        onehot, wn, (((1,), (0,)), ((), ())),
        preferred_element_type=jnp.float32) * 0.5


def kernel(u_e, i_e, pref_w, rel_w, pref_norm_w, norm_w):
    g = jnp.asarray(_gumbel_const())

    grid = (_T // _BT,)
    tok = lambda t: (t, 0)
    fixed = lambda t: (0, 0)
    pre, re, nm = pl.pallas_call(
        _body,
        grid=grid,
        in_specs=[
            pl.BlockSpec((_BT, _D), tok),
            pl.BlockSpec((_BT, _D), tok),
            pl.BlockSpec((_BT, _E), tok),
            pl.BlockSpec((_E, _D), fixed),
            pl.BlockSpec((_E, _D), fixed),
            pl.BlockSpec((_E, _D), fixed),
            pl.BlockSpec((_E, _D), fixed),
        ],
        out_specs=[
            pl.BlockSpec((_BT, _E), tok),
            pl.BlockSpec((_BT, _D), tok),
            pl.BlockSpec((_BT, _D), tok),
        ],
        out_shape=[
            jax.ShapeDtypeStruct((_T, _E), jnp.float32),
            jax.ShapeDtypeStruct((_T, _D), jnp.float32),
            jax.ShapeDtypeStruct((_T, _D), jnp.float32),
        ],
        compiler_params=pltpu.CompilerParams(
            dimension_semantics=("parallel",),
        ),
    )(u_e, i_e, g, pref_w, rel_w, pref_norm_w, norm_w)
    return (pre, re, nm)
